# baseline (device time: 75695 ns/iter reference)
import jax
import jax.numpy as jnp
from jax import lax
from jax.experimental import pallas as pl
from jax.experimental.pallas import tpu as pltpu

N_DEV = 16
STEPS = 4

SCHED_ROWS = [1024, 768, 256]
N_SCHED = len(SCHED_ROWS)


def kernel(t, W):
    M, K = t.shape
    _, N = W.shape
    assert sum(SCHED_ROWS) == M

    step_rows = [[r >> (k + 1) for k in range(STEPS)] for r in SCHED_ROWS]
    sched_base = [sum(SCHED_ROWS[:s]) for s in range(N_SCHED)]
    stage_off = []
    acc_off = 0
    for s in range(N_SCHED):
        offs = []
        for k in range(STEPS):
            offs.append(acc_off)
            acc_off += step_rows[s][k]
        stage_off.append(offs)
    stage_total = acc_off

    def body(t_ref, w_ref, out_ref, redbuf, stage, wbuf,
             rs_send_sems, rs_recv_sems, ag_send_sems, ag_recv_sems):
        my = lax.axis_index("i")
        p = lax.rem(my, 4)
        z = my // 4

        side_x = jnp.minimum(p, 3 - p)
        side_y = p // 2
        side_zl = lax.rem(z, 2)
        side_zh = z // 2
        AX = {
            "X": (side_x, my + 1 - 2 * lax.rem(p, 2)),
            "Y": (side_y, my + 3 - 2 * p),
            "ZL": (side_zl, my + 4 * (1 - 2 * side_zl)),
            "ZH": (side_zh, my + 8 * (1 - 2 * side_zh)),
        }
        ORDERS = [
            ["X", "Y", "ZL", "ZH"],
            ["Y", "ZL", "ZH", "X"],
            ["ZH", "X", "Y", "ZL"],
        ]

        barrier_sem = pltpu.get_barrier_semaphore()
        for ax in ("X", "Y", "ZL", "ZH"):
            pl.semaphore_signal(
                barrier_sem, inc=1,
                device_id=(AX[ax][1],), device_id_type=pl.DeviceIdType.MESH,
            )
        pl.semaphore_wait(barrier_sem, 4)

        redbuf[...] = t_ref[...].astype(jnp.bfloat16)

        send_descs = []

        def rs_copy(s, step, pt_base, rows, pt):
            off = stage_off[s][step]
            return pltpu.make_async_remote_copy(
                src_ref=redbuf.at[pl.ds(pt_base, rows), :],
                dst_ref=stage.at[pl.ds(off, rows), :],
                send_sem=rs_send_sems.at[4 * s + step],
                recv_sem=rs_recv_sems.at[4 * s + step],
                device_id=(pt,),
                device_id_type=pl.DeviceIdType.MESH,
            )

        bases = [jnp.int32(sched_base[s]) for s in range(N_SCHED)]
        lens = list(SCHED_ROWS)

        def rs_send(s, step):
            side, pt = AX[ORDERS[s][step]]
            h = lens[s] // 2
            pt_base = bases[s] + (1 - side) * h
            d = rs_copy(s, step, pt_base, h, pt)
            d.start()
            send_descs.append(d)

        def rs_recv_add(s, step):
            side, pt = AX[ORDERS[s][step]]
            h = lens[s] // 2
            my_base = bases[s] + side * h
            rs_copy(s, step, my_base, h, pt).wait_recv()
            off = stage_off[s][step]
            acc = (
                redbuf[pl.ds(my_base, h), :].astype(jnp.float32)
                + stage[pl.ds(off, h), :].astype(jnp.float32)
            )
            redbuf[pl.ds(my_base, h), :] = acc.astype(jnp.bfloat16)
            bases[s] = my_base
            lens[s] = h

        for s in range(N_SCHED):
            rs_send(s, 0)
        wbuf[...] = w_ref[...].astype(jnp.bfloat16)
        for step in range(1, STEPS):
            for s in range(N_SCHED):
                rs_recv_add(s, step - 1)
                rs_send(s, step)
        for s in range(N_SCHED):
            rs_recv_add(s, STEPS - 1)

        for s in range(N_SCHED):
            fr = SCHED_ROWS[s] // N_DEV
            y = jnp.dot(redbuf[pl.ds(bases[s], fr), :], wbuf[...],
                        preferred_element_type=jnp.float32)
            out_ref[pl.ds(bases[s], fr), :] = y.astype(jnp.bfloat16)

        def ag_copy(s, step, src_base, dst_base, rows, pt):
            return pltpu.make_async_remote_copy(
                src_ref=out_ref.at[pl.ds(src_base, rows), :],
                dst_ref=out_ref.at[pl.ds(dst_base, rows), :],
                send_sem=ag_send_sems.at[4 * s + step],
                recv_sem=ag_recv_sems.at[4 * s + step],
                device_id=(pt,),
                device_id_type=pl.DeviceIdType.MESH,
            )

        cur_b = list(bases)
        cur_l = [SCHED_ROWS[s] // N_DEV for s in range(N_SCHED)]
        for step in range(STEPS):
            sibs = []
            for s in range(N_SCHED):
                side, pt = AX[ORDERS[s][STEPS - 1 - step]]
                parent_b = cur_b[s] - side * cur_l[s]
                sib_b = parent_b + (1 - side) * cur_l[s]
                d = ag_copy(s, step, cur_b[s], cur_b[s], cur_l[s], pt)
                d.start()
                send_descs.append(d)
                sibs.append((sib_b, parent_b, pt))
            for s in range(N_SCHED):
                sib_b, parent_b, pt = sibs[s]
                ag_copy(s, step, sib_b, sib_b, cur_l[s], pt).wait_recv()
                cur_b[s] = parent_b
                cur_l[s] *= 2

        for d in send_descs:
            d.wait_send()

    return pl.pallas_call(
        body,
        out_shape=jax.ShapeDtypeStruct((M, N), jnp.bfloat16),
        in_specs=[
            pl.BlockSpec(memory_space=pltpu.VMEM),
            pl.BlockSpec(memory_space=pltpu.VMEM),
        ],
        out_specs=pl.BlockSpec(memory_space=pltpu.VMEM),
        scratch_shapes=[
            pltpu.VMEM((M, K), jnp.bfloat16),
            pltpu.VMEM((stage_total, K), jnp.bfloat16),
            pltpu.VMEM((K, N), jnp.bfloat16),
            pltpu.SemaphoreType.DMA((16,)),
            pltpu.SemaphoreType.DMA((16,)),
            pltpu.SemaphoreType.DMA((16,)),
            pltpu.SemaphoreType.DMA((16,)),
        ],
        compiler_params=pltpu.CompilerParams(collective_id=0),
    )(t, W)


# device time: 74506 ns/iter; 1.0160x vs baseline; 1.0160x over previous
import jax
import jax.numpy as jnp
from jax import lax
from jax.experimental import pallas as pl
from jax.experimental.pallas import tpu as pltpu

N_DEV = 16
STEPS = 4

_ORDER_A = ["X", "Y", "ZL", "ZH"]
_ORDER_B = ["Y", "ZL", "ZH", "X"]
_ORDER_C = ["ZH", "X", "Y", "ZL"]
SCHED_ORDERS = [_ORDER_A] * 4 + [_ORDER_B] * 3 + [_ORDER_C]
SCHED_ROWS = [256] * len(SCHED_ORDERS)
N_SCHED = len(SCHED_ROWS)


def kernel(t, W):
    M, K = t.shape
    _, N = W.shape
    assert sum(SCHED_ROWS) == M

    step_rows = [[r >> (k + 1) for k in range(STEPS)] for r in SCHED_ROWS]
    sched_base = [sum(SCHED_ROWS[:s]) for s in range(N_SCHED)]
    stage_off = []
    acc_off = 0
    for s in range(N_SCHED):
        offs = []
        for k in range(STEPS):
            offs.append(acc_off)
            acc_off += step_rows[s][k]
        stage_off.append(offs)
    stage_total = acc_off

    def body(t_ref, w_ref, out_ref, redbuf, stage, wbuf,
             rs_send_sems, rs_recv_sems, ag_send_sems, ag_recv_sems):
        my = lax.axis_index("i")
        p = lax.rem(my, 4)
        z = my // 4

        side_x = jnp.minimum(p, 3 - p)
        side_y = p // 2
        side_zl = lax.rem(z, 2)
        side_zh = z // 2
        AX = {
            "X": (side_x, my + 1 - 2 * lax.rem(p, 2)),
            "Y": (side_y, my + 3 - 2 * p),
            "ZL": (side_zl, my + 4 * (1 - 2 * side_zl)),
            "ZH": (side_zh, my + 8 * (1 - 2 * side_zh)),
        }
        ORDERS = SCHED_ORDERS

        barrier_sem = pltpu.get_barrier_semaphore()
        for ax in ("X", "Y", "ZL", "ZH"):
            pl.semaphore_signal(
                barrier_sem, inc=1,
                device_id=(AX[ax][1],), device_id_type=pl.DeviceIdType.MESH,
            )
        pl.semaphore_wait(barrier_sem, 4)

        redbuf[...] = t_ref[...].astype(jnp.bfloat16)

        send_descs = []

        def rs_copy(s, step, pt_base, rows, pt):
            off = stage_off[s][step]
            return pltpu.make_async_remote_copy(
                src_ref=redbuf.at[pl.ds(pt_base, rows), :],
                dst_ref=stage.at[pl.ds(off, rows), :],
                send_sem=rs_send_sems.at[4 * s + step],
                recv_sem=rs_recv_sems.at[4 * s + step],
                device_id=(pt,),
                device_id_type=pl.DeviceIdType.MESH,
            )

        bases = [jnp.int32(sched_base[s]) for s in range(N_SCHED)]
        lens = list(SCHED_ROWS)

        def rs_send(s, step):
            side, pt = AX[ORDERS[s][step]]
            h = lens[s] // 2
            pt_base = bases[s] + (1 - side) * h
            d = rs_copy(s, step, pt_base, h, pt)
            d.start()
            send_descs.append(d)

        def rs_recv_add(s, step):
            side, pt = AX[ORDERS[s][step]]
            h = lens[s] // 2
            my_base = bases[s] + side * h
            rs_copy(s, step, my_base, h, pt).wait_recv()
            off = stage_off[s][step]
            acc = (
                redbuf[pl.ds(my_base, h), :].astype(jnp.float32)
                + stage[pl.ds(off, h), :].astype(jnp.float32)
            )
            redbuf[pl.ds(my_base, h), :] = acc.astype(jnp.bfloat16)
            bases[s] = my_base
            lens[s] = h

        for s in range(N_SCHED):
            rs_send(s, 0)
        wbuf[...] = w_ref[...].astype(jnp.bfloat16)
        for step in range(1, STEPS):
            for s in range(N_SCHED):
                rs_recv_add(s, step - 1)
                rs_send(s, step)
        for s in range(N_SCHED):
            rs_recv_add(s, STEPS - 1)

        for s in range(N_SCHED):
            fr = SCHED_ROWS[s] // N_DEV
            y = jnp.dot(redbuf[pl.ds(bases[s], fr), :], wbuf[...],
                        preferred_element_type=jnp.float32)
            out_ref[pl.ds(bases[s], fr), :] = y.astype(jnp.bfloat16)

        def ag_copy(s, step, src_base, dst_base, rows, pt):
            return pltpu.make_async_remote_copy(
                src_ref=out_ref.at[pl.ds(src_base, rows), :],
                dst_ref=out_ref.at[pl.ds(dst_base, rows), :],
                send_sem=ag_send_sems.at[4 * s + step],
                recv_sem=ag_recv_sems.at[4 * s + step],
                device_id=(pt,),
                device_id_type=pl.DeviceIdType.MESH,
            )

        cur_b = list(bases)
        cur_l = [SCHED_ROWS[s] // N_DEV for s in range(N_SCHED)]
        for step in range(STEPS):
            sibs = []
            for s in range(N_SCHED):
                side, pt = AX[ORDERS[s][STEPS - 1 - step]]
                parent_b = cur_b[s] - side * cur_l[s]
                sib_b = parent_b + (1 - side) * cur_l[s]
                d = ag_copy(s, step, cur_b[s], cur_b[s], cur_l[s], pt)
                d.start()
                send_descs.append(d)
                sibs.append((sib_b, parent_b, pt))
            for s in range(N_SCHED):
                sib_b, parent_b, pt = sibs[s]
                ag_copy(s, step, sib_b, sib_b, cur_l[s], pt).wait_recv()
                cur_b[s] = parent_b
                cur_l[s] *= 2

        for d in send_descs:
            d.wait_send()

    return pl.pallas_call(
        body,
        out_shape=jax.ShapeDtypeStruct((M, N), jnp.bfloat16),
        in_specs=[
            pl.BlockSpec(memory_space=pltpu.VMEM),
            pl.BlockSpec(memory_space=pltpu.VMEM),
        ],
        out_specs=pl.BlockSpec(memory_space=pltpu.VMEM),
        scratch_shapes=[
            pltpu.VMEM((M, K), jnp.bfloat16),
            pltpu.VMEM((stage_total, K), jnp.bfloat16),
            pltpu.VMEM((K, N), jnp.bfloat16),
            pltpu.SemaphoreType.DMA((4 * N_SCHED,)),
            pltpu.SemaphoreType.DMA((4 * N_SCHED,)),
            pltpu.SemaphoreType.DMA((4 * N_SCHED,)),
            pltpu.SemaphoreType.DMA((4 * N_SCHED,)),
        ],
        compiler_params=pltpu.CompilerParams(collective_id=0),
    )(t, W)
